# baseline (speedup 1.0000x reference)
import functools

import jax
import jax.numpy as jnp
from jax import lax
from jax.experimental import pallas as pl
from jax.experimental.pallas import tpu as pltpu
from jax.experimental.pallas import tpu_sc as plsc

BATCH = 16384
DIM = 32
CHUNK = 128


def _sc_gather4(user2d, item2d, gu_tab, gi_tab, mu_tab, mi_tab):
    info = plsc.get_sparse_core_info()
    nc, ns = info.num_cores, info.num_subcores
    nw = nc * ns
    b_per_w = BATCH // nw
    n_chunks = b_per_w // CHUNK
    mesh = plsc.VectorSubcoreMesh(core_axis_name="c", subcore_axis_name="s")
    out_sds = jax.ShapeDtypeStruct((BATCH, DIM), jnp.float32)

    @functools.partial(
        pl.kernel,
        mesh=mesh,
        out_type=[out_sds] * 4,
        compiler_params=pltpu.CompilerParams(use_tc_tiling_on_sc=False),
        scratch_types=[
            pltpu.VMEM((n_chunks, CHUNK), jnp.int32),
            pltpu.VMEM((n_chunks, CHUNK), jnp.int32),
            pltpu.VMEM((b_per_w, DIM), jnp.float32),
            pltpu.VMEM((b_per_w, DIM), jnp.float32),
            pltpu.VMEM((b_per_w, DIM), jnp.float32),
            pltpu.VMEM((b_per_w, DIM), jnp.float32),
            pltpu.SemaphoreType.DMA,
        ],
    )
    def k(user_hbm, item_hbm, gu_hbm, gi_hbm, mu_hbm, mi_hbm,
          out_gu, out_gi, out_mu, out_mi,
          idx_u, idx_i, bgu, bgi, bmu, bmi, sem):
        wid = lax.axis_index("s") * nc + lax.axis_index("c")
        base = wid * b_per_w
        pltpu.sync_copy(user_hbm.at[pl.ds(wid * n_chunks, n_chunks)], idx_u)
        pltpu.sync_copy(item_hbm.at[pl.ds(wid * n_chunks, n_chunks)], idx_i)
        copies = []
        for j in range(n_chunks):
            dst = pl.ds(j * CHUNK, CHUNK)
            copies.append(pltpu.async_copy(gu_hbm.at[idx_u.at[j]], bgu.at[dst], sem))
            copies.append(pltpu.async_copy(gi_hbm.at[idx_i.at[j]], bgi.at[dst], sem))
            copies.append(pltpu.async_copy(mu_hbm.at[idx_u.at[j]], bmu.at[dst], sem))
            copies.append(pltpu.async_copy(mi_hbm.at[idx_i.at[j]], bmi.at[dst], sem))
        for c in copies:
            c.wait()
        dst = pl.ds(base, b_per_w)
        pltpu.sync_copy(bgu, out_gu.at[dst])
        pltpu.sync_copy(bgi, out_gi.at[dst])
        pltpu.sync_copy(bmu, out_mu.at[dst])
        pltpu.sync_copy(bmi, out_mi.at[dst])

    return k(user2d, item2d, gu_tab, gi_tab, mu_tab, mi_tab)


def _tc_mlp(gu, gi, mu, mi, w1u, w1i, w2t, wfg, wfh, b1r, b2r, bfr):
    bm = 2048
    grid = (BATCH // bm,)

    def body(gu_ref, gi_ref, mu_ref, mi_ref, w1u_ref, w1i_ref, w2_ref,
             wfg_ref, wfh_ref, b1_ref, b2_ref, bf_ref, out_ref):
        h1 = (jnp.dot(mu_ref[...], w1u_ref[...], preferred_element_type=jnp.float32)
              + jnp.dot(mi_ref[...], w1i_ref[...], preferred_element_type=jnp.float32)
              + b1_ref[...])
        h1 = jnp.maximum(h1, 0.0)
        h2 = jnp.maximum(
            jnp.dot(h1, w2_ref[...], preferred_element_type=jnp.float32) + b2_ref[...],
            0.0)
        g = gu_ref[...] * gi_ref[...]
        out = (jnp.sum(g * wfg_ref[...], axis=1, keepdims=True)
               + jnp.sum(h2 * wfh_ref[...], axis=1, keepdims=True)
               + bf_ref[...])
        out_ref[...] = out

    full = lambda shape: pl.BlockSpec(shape, lambda i: (0, 0))
    return pl.pallas_call(
        body,
        grid=grid,
        in_specs=[
            pl.BlockSpec((bm, DIM), lambda i: (i, 0)),
            pl.BlockSpec((bm, DIM), lambda i: (i, 0)),
            pl.BlockSpec((bm, DIM), lambda i: (i, 0)),
            pl.BlockSpec((bm, DIM), lambda i: (i, 0)),
            full((DIM, 64)), full((DIM, 64)), full((64, DIM)),
            full((1, DIM)), full((1, DIM)),
            full((1, 64)), full((1, DIM)), full((1, 1)),
        ],
        out_specs=pl.BlockSpec((bm, 1), lambda i: (i, 0)),
        out_shape=jax.ShapeDtypeStruct((BATCH, 1), jnp.float32),
    )(gu, gi, mu, mi, w1u, w1i, w2t, wfg, wfh, b1r, b2r, bfr)


def kernel(user, item, gmf_user_emb, gmf_item_emb, mlp_user_emb, mlp_item_emb,
           W1, b1, W2, b2, Wf, bf):
    user2d = user.astype(jnp.int32).reshape(-1, CHUNK)
    item2d = item.astype(jnp.int32).reshape(-1, CHUNK)
    gu, gi, mu, mi = _sc_gather4(
        user2d, item2d, gmf_user_emb, gmf_item_emb, mlp_user_emb, mlp_item_emb)
    w1u = W1[:, :DIM].T
    w1i = W1[:, DIM:].T
    w2t = W2.T
    wfg = Wf[:, :DIM]
    wfh = Wf[:, DIM:]
    b1r = b1.reshape(1, 64)
    b2r = b2.reshape(1, DIM)
    bfr = bf.reshape(1, 1)
    out = _tc_mlp(gu, gi, mu, mi, w1u, w1i, w2t, wfg, wfh, b1r, b2r, bfr)
    return out[:, 0]


# speedup vs baseline: 1.0735x; 1.0735x over previous
import functools

import jax
import jax.numpy as jnp
from jax import lax
from jax.experimental import pallas as pl
from jax.experimental.pallas import tpu as pltpu
from jax.experimental.pallas import tpu_sc as plsc

BATCH = 16384
DIM = 32
V = 100000
W16 = 16
RPF = V // W16
CHUNK = 128


def _sc_gather4(user2d, item2d, gu16, gi16, mu16, mi16):
    info = plsc.get_sparse_core_info()
    nc, ns = info.num_cores, info.num_subcores
    nw = nc * ns
    b_per_w = BATCH // nw
    n_chunks = b_per_w // CHUNK
    mesh = plsc.VectorSubcoreMesh(core_axis_name="c", subcore_axis_name="s")
    out_sds = jax.ShapeDtypeStruct((4 * DIM, BATCH), jnp.float32)
    nvreg = CHUNK // 16

    @functools.partial(
        pl.kernel,
        mesh=mesh,
        out_type=out_sds,
        compiler_params=pltpu.CompilerParams(
            needs_layout_passes=False, use_tc_tiling_on_sc=False),
        scratch_types=[
            pltpu.VMEM((n_chunks, CHUNK), jnp.int32),
            pltpu.VMEM((n_chunks, CHUNK), jnp.int32),
            pltpu.VMEM((n_chunks, CHUNK), jnp.int32),
            pltpu.VMEM((n_chunks, CHUNK), jnp.int32),
            pltpu.VMEM((n_chunks, CHUNK), jnp.int32),
            pltpu.VMEM((n_chunks, CHUNK), jnp.int32),
            pltpu.VMEM((4, CHUNK), jnp.int32),
            pltpu.VMEM((4, CHUNK), jnp.int32),
            pltpu.VMEM((DIM, CHUNK, W16), jnp.float32),
            pltpu.VMEM((2, DIM, CHUNK), jnp.float32),
            pltpu.SemaphoreType.DMA,
            pltpu.SemaphoreType.DMA,
        ],
    )
    def k(user_hbm, item_hbm, gu_hbm, gi_hbm, mu_hbm, mi_hbm,
          out_x, idx_u, idx_i, j0_u, j0_i, l4_u, l4_i,
          lst_u, lst_i, stg, mini, sem, osem):
        wid = lax.axis_index("s") * nc + lax.axis_index("c")
        pltpu.sync_copy(user_hbm.at[pl.ds(wid * n_chunks, n_chunks)], idx_u)
        pltpu.sync_copy(item_hbm.at[pl.ds(wid * n_chunks, n_chunks)], idx_i)
        for c in range(n_chunks):
            for g in range(nvreg):
                s = pl.ds(g * 16, 16)
                vu = idx_u[c, s]
                vi = idx_i[c, s]
                j0_u[c, s] = jax.lax.shift_right_logical(vu, 4)
                j0_i[c, s] = jax.lax.shift_right_logical(vi, 4)
                l4_u[c, s] = jax.lax.bitwise_and(vu, 15)
                l4_i[c, s] = jax.lax.bitwise_and(vi, 15)

        tables = (gu_hbm, gi_hbm, mu_hbm, mi_hbm)
        riota = lax.iota(jnp.int32, 16)

        def chunk_body(c, carry):
            for g in range(nvreg):
                s = pl.ds(g * 16, 16)
                ju = j0_u[c, s]
                ji = j0_i[c, s]
                for v in range(4):
                    lst_u[v, s] = ju + (v * RPF)
                    lst_i[v, s] = ji + (v * RPF)
            for t in range(4):
                lst = lst_u if t in (0, 2) else lst_i
                l4 = l4_u if t in (0, 2) else l4_i
                copies = []
                for f in range(DIM):
                    src = tables[t].at[pl.ds((f // 4) * (4 * RPF), 4 * RPF)]
                    copies.append(pltpu.async_copy(
                        src.at[lst.at[f % 4]], stg.at[f], sem))
                for cp in copies:
                    cp.wait()
                mb = mini.at[(t + c) % 2]
                for g in range(nvreg):
                    rows = riota + (g * 16)
                    lanes = l4[c, pl.ds(g * 16, 16)]
                    for f in range(DIM):
                        got = plsc.load_gather(stg.at[f], [rows, lanes])
                        mb[f, pl.ds(g * 16, 16)] = got
                pltpu.async_copy(
                    mb,
                    out_x.at[pl.ds(t * DIM, DIM),
                             pl.ds(wid * b_per_w + c * CHUNK, CHUNK)],
                    osem).wait()
            return carry

        lax.fori_loop(0, n_chunks, chunk_body, 0)

    return k(user2d, item2d, gu16, gi16, mu16, mi16)


def _tc_mlp(x, W1, b1c, W2, b2c, Wf, bfc):
    bm = 4096
    grid = (BATCH // bm,)

    def body(x_ref, w1_ref, b1_ref, w2_ref, b2_ref, wf_ref, bf_ref, out_ref):
        xb = x_ref[...]
        g = xb[0:DIM, :] * xb[DIM:2 * DIM, :]
        h1 = jnp.maximum(
            jnp.dot(w1_ref[...], xb[2 * DIM:, :],
                    preferred_element_type=jnp.float32) + b1_ref[...], 0.0)
        h2 = jnp.maximum(
            jnp.dot(w2_ref[...], h1,
                    preferred_element_type=jnp.float32) + b2_ref[...], 0.0)
        comb = jnp.concatenate([g, h2], axis=0)
        out_ref[...] = (jnp.dot(wf_ref[...], comb,
                                preferred_element_type=jnp.float32)
                        + bf_ref[...])

    full = lambda shape: pl.BlockSpec(shape, lambda i: tuple(0 for _ in shape))
    return pl.pallas_call(
        body,
        grid=grid,
        in_specs=[
            pl.BlockSpec((4 * DIM, bm), lambda i: (0, i)),
            full((64, 64)), full((64, 1)),
            full((DIM, 64)), full((DIM, 1)),
            full((1, 64)), full((1, 1)),
        ],
        out_specs=pl.BlockSpec((1, bm), lambda i: (0, i)),
        out_shape=jax.ShapeDtypeStruct((1, BATCH), jnp.float32),
    )(x, W1, b1c, W2, b2c, Wf, bfc)


def kernel(user, item, gmf_user_emb, gmf_item_emb, mlp_user_emb, mlp_item_emb,
           W1, b1, W2, b2, Wf, bf):
    user2d = user.astype(jnp.int32).reshape(-1, CHUNK)
    item2d = item.astype(jnp.int32).reshape(-1, CHUNK)
    x = _sc_gather4(user2d, item2d,
                    gmf_user_emb.T.reshape(DIM * RPF, W16),
                    gmf_item_emb.T.reshape(DIM * RPF, W16),
                    mlp_user_emb.T.reshape(DIM * RPF, W16),
                    mlp_item_emb.T.reshape(DIM * RPF, W16))
    out = _tc_mlp(x, W1, b1.reshape(64, 1), W2, b2.reshape(DIM, 1),
                  Wf, bf.reshape(1, 1))
    return out[0]


# speedup vs baseline: 1.5722x; 1.4645x over previous
import functools

import jax
import jax.numpy as jnp
from jax import lax
from jax.experimental import pallas as pl
from jax.experimental.pallas import tpu as pltpu
from jax.experimental.pallas import tpu_sc as plsc

BATCH = 16384
DIM = 32
V = 100000
CHUNK = 128
TBLK = 2048


def _tc_transpose4(gu_t, mu_t, gi_t, mi_t):
    grid = (pl.cdiv(V, TBLK),)

    def body(gu_ref, mu_ref, gi_ref, mi_ref, out_ref):
        out_ref[...] = jnp.concatenate(
            [gu_ref[...].T, mu_ref[...].T, gi_ref[...].T, mi_ref[...].T],
            axis=1)

    spec = pl.BlockSpec((DIM, TBLK), lambda i: (0, i))
    return pl.pallas_call(
        body,
        grid=grid,
        in_specs=[spec, spec, spec, spec],
        out_specs=pl.BlockSpec((TBLK, 4 * DIM), lambda i: (i, 0)),
        out_shape=jax.ShapeDtypeStruct((V, 4 * DIM), jnp.float32),
    )(gu_t, mu_t, gi_t, mi_t)


def _sc_gather(user2d, item2d, t4):
    info = plsc.get_sparse_core_info()
    nc, ns = info.num_cores, info.num_subcores
    nw = nc * ns
    b_per_w = BATCH // nw
    n_chunks = b_per_w // CHUNK
    mesh = plsc.VectorSubcoreMesh(core_axis_name="c", subcore_axis_name="s")
    out_sds = jax.ShapeDtypeStruct((BATCH, 4 * DIM), jnp.float32)

    @functools.partial(
        pl.kernel,
        mesh=mesh,
        out_type=out_sds,
        compiler_params=pltpu.CompilerParams(use_tc_tiling_on_sc=False),
        scratch_types=[
            pltpu.VMEM((n_chunks, CHUNK), jnp.int32),
            pltpu.VMEM((n_chunks, CHUNK), jnp.int32),
            pltpu.VMEM((2, CHUNK, 4 * DIM), jnp.float32),
            pltpu.SemaphoreType.DMA,
            pltpu.SemaphoreType.DMA,
        ],
    )
    def k(user_hbm, item_hbm, t4_hbm, out_x, idx_u, idx_i, stg, sem, osem):
        wid = lax.axis_index("s") * nc + lax.axis_index("c")
        pltpu.sync_copy(user_hbm.at[pl.ds(wid * n_chunks, n_chunks)], idx_u)
        pltpu.sync_copy(item_hbm.at[pl.ds(wid * n_chunks, n_chunks)], idx_i)
        for c in range(n_chunks):
            cu = pltpu.async_copy(t4_hbm.at[idx_u.at[c]], stg.at[0], sem)
            ci = pltpu.async_copy(t4_hbm.at[idx_i.at[c]], stg.at[1], sem)
            rows = pl.ds(wid * b_per_w + c * CHUNK, CHUNK)
            cu.wait()
            ou = pltpu.async_copy(
                stg.at[0, slice(None), pl.ds(0, 2 * DIM)],
                out_x.at[rows, pl.ds(0, 2 * DIM)], osem)
            ci.wait()
            oi = pltpu.async_copy(
                stg.at[1, slice(None), pl.ds(2 * DIM, 2 * DIM)],
                out_x.at[rows, pl.ds(2 * DIM, 2 * DIM)], osem)
            ou.wait()
            oi.wait()

    return k(user2d, item2d, t4)


def _tc_mlp(x, w1u_t, w1i_t, w2t, wfg, wfh, b1r, b2r, bfr):
    bm = 2048
    grid = (BATCH // bm,)

    def body(x_ref, w1u_ref, w1i_ref, w2_ref, wfg_ref, wfh_ref,
             b1_ref, b2_ref, bf_ref, out_ref):
        xb = x_ref[...]
        g = xb[:, 0:DIM] * xb[:, 2 * DIM:3 * DIM]
        h1 = jnp.maximum(
            jnp.dot(xb[:, DIM:2 * DIM], w1u_ref[...],
                    preferred_element_type=jnp.float32)
            + jnp.dot(xb[:, 3 * DIM:], w1i_ref[...],
                      preferred_element_type=jnp.float32)
            + b1_ref[...], 0.0)
        h2 = jnp.maximum(
            jnp.dot(h1, w2_ref[...], preferred_element_type=jnp.float32)
            + b2_ref[...], 0.0)
        out_ref[...] = (
            jnp.sum(g * wfg_ref[...], axis=1, keepdims=True)
            + jnp.sum(h2 * wfh_ref[...], axis=1, keepdims=True)
            + bf_ref[...])

    full = lambda shape: pl.BlockSpec(shape, lambda i: tuple(0 for _ in shape))
    return pl.pallas_call(
        body,
        grid=grid,
        in_specs=[
            pl.BlockSpec((bm, 4 * DIM), lambda i: (i, 0)),
            full((DIM, 64)), full((DIM, 64)), full((64, DIM)),
            full((1, DIM)), full((1, DIM)),
            full((1, 64)), full((1, DIM)), full((1, 1)),
        ],
        out_specs=pl.BlockSpec((bm, 1), lambda i: (i, 0)),
        out_shape=jax.ShapeDtypeStruct((BATCH, 1), jnp.float32),
    )(x, w1u_t, w1i_t, w2t, wfg, wfh, b1r, b2r, bfr)


def kernel(user, item, gmf_user_emb, gmf_item_emb, mlp_user_emb, mlp_item_emb,
           W1, b1, W2, b2, Wf, bf):
    user2d = user.astype(jnp.int32).reshape(-1, CHUNK)
    item2d = item.astype(jnp.int32).reshape(-1, CHUNK)
    t4 = _tc_transpose4(gmf_user_emb.T, mlp_user_emb.T,
                        gmf_item_emb.T, mlp_item_emb.T)
    x = _sc_gather(user2d, item2d, t4)
    out = _tc_mlp(x, W1[:, :DIM].T, W1[:, DIM:].T, W2.T,
                  Wf[:, :DIM], Wf[:, DIM:],
                  b1.reshape(1, 64), b2.reshape(1, DIM), bf.reshape(1, 1))
    return out[:, 0]


# speedup vs baseline: 1.5783x; 1.0039x over previous
import functools

import jax
import jax.numpy as jnp
from jax import lax
from jax.experimental import pallas as pl
from jax.experimental.pallas import tpu as pltpu
from jax.experimental.pallas import tpu_sc as plsc

BATCH = 16384
DIM = 32
V = 100000
CHUNK = 128
TBLK = 4096


def _tc_transpose4(gu_t, mu_t, gi_t, mi_t):
    grid = (pl.cdiv(V, TBLK),)

    def body(gu_ref, mu_ref, gi_ref, mi_ref, out_ref):
        eye = jnp.eye(DIM, dtype=jnp.float32)
        dn = (((0,), (0,)), ((), ()))

        def tr(ref):
            return jax.lax.dot_general(ref[...], eye, dn,
                                       preferred_element_type=jnp.float32)

        out_ref[...] = jnp.concatenate(
            [tr(gu_ref), tr(mu_ref), tr(gi_ref), tr(mi_ref)], axis=1)

    spec = pl.BlockSpec((DIM, TBLK), lambda i: (0, i))
    return pl.pallas_call(
        body,
        grid=grid,
        in_specs=[spec, spec, spec, spec],
        out_specs=pl.BlockSpec((TBLK, 4 * DIM), lambda i: (i, 0)),
        out_shape=jax.ShapeDtypeStruct((V, 4 * DIM), jnp.float32),
    )(gu_t, mu_t, gi_t, mi_t)


def _sc_gather(user2d, item2d, t4):
    info = plsc.get_sparse_core_info()
    nc, ns = info.num_cores, info.num_subcores
    nw = nc * ns
    b_per_w = BATCH // nw
    n_chunks = b_per_w // CHUNK
    mesh = plsc.VectorSubcoreMesh(core_axis_name="c", subcore_axis_name="s")
    out_sds = jax.ShapeDtypeStruct((BATCH, 4 * DIM), jnp.float32)

    @functools.partial(
        pl.kernel,
        mesh=mesh,
        out_type=out_sds,
        compiler_params=pltpu.CompilerParams(use_tc_tiling_on_sc=False),
        scratch_types=[
            pltpu.VMEM((n_chunks, CHUNK), jnp.int32),
            pltpu.VMEM((n_chunks, CHUNK), jnp.int32),
            pltpu.VMEM((2, CHUNK, 4 * DIM), jnp.float32),
            pltpu.SemaphoreType.DMA,
            pltpu.SemaphoreType.DMA,
        ],
    )
    def k(user_hbm, item_hbm, t4_hbm, out_x, idx_u, idx_i, stg, sem, osem):
        wid = lax.axis_index("s") * nc + lax.axis_index("c")
        pltpu.sync_copy(user_hbm.at[pl.ds(wid * n_chunks, n_chunks)], idx_u)
        pltpu.sync_copy(item_hbm.at[pl.ds(wid * n_chunks, n_chunks)], idx_i)
        for c in range(n_chunks):
            cu = pltpu.async_copy(t4_hbm.at[idx_u.at[c]], stg.at[0], sem)
            ci = pltpu.async_copy(t4_hbm.at[idx_i.at[c]], stg.at[1], sem)
            rows = pl.ds(wid * b_per_w + c * CHUNK, CHUNK)
            cu.wait()
            ou = pltpu.async_copy(
                stg.at[0, slice(None), pl.ds(0, 2 * DIM)],
                out_x.at[rows, pl.ds(0, 2 * DIM)], osem)
            ci.wait()
            oi = pltpu.async_copy(
                stg.at[1, slice(None), pl.ds(2 * DIM, 2 * DIM)],
                out_x.at[rows, pl.ds(2 * DIM, 2 * DIM)], osem)
            ou.wait()
            oi.wait()

    return k(user2d, item2d, t4)


def _tc_mlp(x, w1u_t, w1i_t, w2t, wfg, wfh, b1r, b2r, bfr):
    bm = 2048
    grid = (BATCH // bm,)

    def body(x_ref, w1u_ref, w1i_ref, w2_ref, wfg_ref, wfh_ref,
             b1_ref, b2_ref, bf_ref, out_ref):
        xb = x_ref[...]
        g = xb[:, 0:DIM] * xb[:, 2 * DIM:3 * DIM]
        h1 = jnp.maximum(
            jnp.dot(xb[:, DIM:2 * DIM], w1u_ref[...],
                    preferred_element_type=jnp.float32)
            + jnp.dot(xb[:, 3 * DIM:], w1i_ref[...],
                      preferred_element_type=jnp.float32)
            + b1_ref[...], 0.0)
        h2 = jnp.maximum(
            jnp.dot(h1, w2_ref[...], preferred_element_type=jnp.float32)
            + b2_ref[...], 0.0)
        out_ref[...] = (
            jnp.sum(g * wfg_ref[...], axis=1, keepdims=True)
            + jnp.sum(h2 * wfh_ref[...], axis=1, keepdims=True)
            + bf_ref[...])

    full = lambda shape: pl.BlockSpec(shape, lambda i: tuple(0 for _ in shape))
    return pl.pallas_call(
        body,
        grid=grid,
        in_specs=[
            pl.BlockSpec((bm, 4 * DIM), lambda i: (i, 0)),
            full((DIM, 64)), full((DIM, 64)), full((64, DIM)),
            full((1, DIM)), full((1, DIM)),
            full((1, 64)), full((1, DIM)), full((1, 1)),
        ],
        out_specs=pl.BlockSpec((bm, 1), lambda i: (i, 0)),
        out_shape=jax.ShapeDtypeStruct((BATCH, 1), jnp.float32),
    )(x, w1u_t, w1i_t, w2t, wfg, wfh, b1r, b2r, bfr)


def kernel(user, item, gmf_user_emb, gmf_item_emb, mlp_user_emb, mlp_item_emb,
           W1, b1, W2, b2, Wf, bf):
    user2d = user.astype(jnp.int32).reshape(-1, CHUNK)
    item2d = item.astype(jnp.int32).reshape(-1, CHUNK)
    t4 = _tc_transpose4(gmf_user_emb.T, mlp_user_emb.T,
                        gmf_item_emb.T, mlp_item_emb.T)
    x = _sc_gather(user2d, item2d, t4)
    out = _tc_mlp(x, W1[:, :DIM].T, W1[:, DIM:].T, W2.T,
                  Wf[:, :DIM], Wf[:, DIM:],
                  b1.reshape(1, 64), b2.reshape(1, DIM), bf.reshape(1, 1))
    return out[:, 0]
